# baseline (speedup 1.0000x reference)
import functools

import jax
import jax.numpy as jnp
from jax import lax
from jax.experimental import pallas as pl
from jax.experimental.pallas import tpu as pltpu
from jax.experimental.pallas import tpu_sc as plsc

NUM_ENTITY = 1000000
DIM = 64
BATCH = 16384

_NC = 2
_NS = 16
_NW = _NC * _NS
_BPW = BATCH // _NW
_CHUNK = 128
_NCHUNK = _BPW // _CHUNK
_EB = 8192


def _sc_body(sidx_hbm, oidx_hbm, fused_hbm, w_hbm, out_hbm,
             sidx_v, oidx_v, s_v, o_v, w_v, out_v, sem):
    wid = lax.axis_index("s") * _NC + lax.axis_index("c")
    base = wid * _BPW
    lane = lax.iota(jnp.int32, 16)
    perms = [lax.bitwise_xor(lane, jnp.int32(k)) for k in (1, 2, 4, 8)]

    pltpu.sync_copy(sidx_hbm.at[pl.ds(base, _BPW)], sidx_v)
    pltpu.sync_copy(oidx_hbm.at[pl.ds(base, _BPW)], oidx_v)
    pltpu.sync_copy(w_hbm, w_v)

    w0 = [w_v[pl.ds(q * 16, 16)] for q in range(4)]
    w1 = [w_v[pl.ds(DIM + q * 16, 16)] for q in range(4)]

    himask = jnp.full((16,), jnp.int32(-65536))

    def re_f32(x):
        return lax.bitcast_convert_type(lax.shift_left(x, 16), jnp.float32)

    def im_f32(x):
        return lax.bitcast_convert_type(lax.bitwise_and(x, himask), jnp.float32)

    for c in range(_NCHUNK):
        co = c * _CHUNK
        s_slice = sidx_v.at[pl.ds(co, _CHUNK)]
        o_slice = oidx_v.at[pl.ds(co, _CHUNK)]
        cp0 = pltpu.make_async_copy(fused_hbm.at[s_slice], s_v, sem)
        cp1 = pltpu.make_async_copy(fused_hbm.at[o_slice], o_v, sem)
        cp0.start(); cp1.start()
        cp0.wait(); cp1.wait()

        def group_body(g, _, co=co):
            out_acc = jnp.zeros((16,), jnp.float32)
            for j in range(16):
                r = g * 16 + j
                acc = None
                for q in range(4):
                    sl = pl.ds(q * 16, 16)
                    s_w = s_v[r, sl]
                    o_w = o_v[r, sl]
                    s_r = re_f32(s_w)
                    s_i = im_f32(s_w)
                    o_r = re_f32(o_w)
                    o_i = im_f32(o_w)
                    a = o_r * w0[q] + o_i * w1[q]
                    b = o_i * w0[q] - o_r * w1[q]
                    t = s_r * a + s_i * b
                    acc = t if acc is None else acc + t
                for p in perms:
                    acc = acc + acc[p]
                out_acc = jnp.where(lane == j, acc, out_acc)
            out_v[pl.ds(co + g * 16, 16)] = out_acc
            return 0

        lax.fori_loop(0, _CHUNK // 16, group_body, 0)

    pltpu.sync_copy(out_v, out_hbm.at[pl.ds(base, _BPW)])


@jax.jit
def _run(s_idx, o_idx, fused_i32, w_flat):
    mesh = plsc.VectorSubcoreMesh(core_axis_name="c", subcore_axis_name="s")
    f = pl.kernel(
        _sc_body,
        out_type=jax.ShapeDtypeStruct((BATCH,), jnp.float32),
        mesh=mesh,
        scratch_types=[
            pltpu.VMEM((_BPW,), jnp.int32),
            pltpu.VMEM((_BPW,), jnp.int32),
            pltpu.VMEM((_CHUNK, DIM), jnp.int32),
            pltpu.VMEM((_CHUNK, DIM), jnp.int32),
            pltpu.VMEM((2 * DIM,), jnp.float32),
            pltpu.VMEM((_BPW,), jnp.float32),
            pltpu.SemaphoreType.DMA,
        ],
        compiler_params=pltpu.CompilerParams(use_tc_tiling_on_sc=False),
    )
    return f(s_idx, o_idx, fused_i32, w_flat)


def _rne_bf16_bits(bits):
    lsb = lax.bitwise_and(lax.shift_right_logical(bits, 16), 1)
    rounded = bits + 0x7FFF + lsb
    return lax.shift_right_logical(rounded, 16)


def _fuse_body(re_ref, im_ref, out_ref):
    x = jnp.concatenate([re_ref[...], im_ref[...]], axis=0)
    ident = jnp.eye(2 * DIM, dtype=jnp.float32)
    y = jax.lax.dot_general(
        x, ident, (((0,), (0,)), ((), ())),
        preferred_element_type=jnp.float32)
    bits = lax.bitcast_convert_type(y, jnp.int32)
    lo = _rne_bf16_bits(bits[:, 0:DIM])
    hi = _rne_bf16_bits(bits[:, DIM:2 * DIM])
    out_ref[...] = lax.bitwise_or(lo, lax.shift_left(hi, 16))


@jax.jit
def _fuse(re_t, im_t):
    grid = (NUM_ENTITY + _EB - 1) // _EB
    return pl.pallas_call(
        _fuse_body,
        grid=(grid,),
        in_specs=[
            pl.BlockSpec((DIM, _EB), lambda j: (0, j)),
            pl.BlockSpec((DIM, _EB), lambda j: (0, j)),
        ],
        out_specs=pl.BlockSpec((_EB, DIM), lambda j: (j, 0)),
        out_shape=jax.ShapeDtypeStruct((NUM_ENTITY, DIM), jnp.int32),
    )(re_t, im_t)


def kernel(idxs, emb_re, emb_im, w):
    idxs = idxs.astype(jnp.int32)
    s_idx = idxs[:, 0]
    o_idx = idxs[:, 1]
    fused = _fuse(emb_re.T, emb_im.T)
    w_flat = w.reshape(-1)
    return _run(s_idx, o_idx, fused, w_flat)


# speedup vs baseline: 2.2101x; 2.2101x over previous
import functools

import jax
import jax.numpy as jnp
from jax import lax
from jax.experimental import pallas as pl
from jax.experimental.pallas import tpu as pltpu
from jax.experimental.pallas import tpu_sc as plsc

NUM_ENTITY = 1000000
DIM = 64
BATCH = 16384

_NC = 2
_NS = 16
_NW = _NC * _NS
_BPW = BATCH // _NW
_CHUNK = 128
_NCHUNK = _BPW // _CHUNK


def _sc_body(sidx_hbm, oidx_hbm, fused_hbm, w_hbm, out_hbm,
             sidx_v, oidx_v, s_v, o_v, w_v, out_v, sem):
    wid = lax.axis_index("s") * _NC + lax.axis_index("c")
    base = wid * _BPW

    pltpu.sync_copy(sidx_hbm.at[pl.ds(base, _BPW)], sidx_v)
    pltpu.sync_copy(oidx_hbm.at[pl.ds(base, _BPW)], oidx_v)
    pltpu.sync_copy(w_hbm, w_v)

    w0 = [w_v[pl.ds(q * 16, 16)] for q in range(4)]
    w1 = [w_v[pl.ds(DIM + q * 16, 16)] for q in range(4)]

    lane = lax.iota(jnp.int32, 16)
    perms = [lax.bitwise_xor(lane, jnp.int32(k)) for k in (1, 2, 4, 8)]

    def start_chunk(c):
        co = c * _CHUNK
        s_slice = sidx_v.at[pl.ds(co, _CHUNK)]
        o_slice = oidx_v.at[pl.ds(co, _CHUNK)]
        cp0 = pltpu.make_async_copy(fused_hbm.at[s_slice], s_v.at[c % 2], sem)
        cp1 = pltpu.make_async_copy(fused_hbm.at[o_slice], o_v.at[c % 2], sem)
        cp0.start(); cp1.start()
        return cp0, cp1

    pend = start_chunk(0)
    for c in range(_NCHUNK):
        co = c * _CHUNK
        pend[0].wait(); pend[1].wait()
        if c + 1 < _NCHUNK:
            pend = start_chunk(c + 1)
        sbuf = s_v.at[c % 2]
        obuf = o_v.at[c % 2]

        def group_body(g, _, co=co, sbuf=sbuf, obuf=obuf):
            out_acc = jnp.zeros((16,), jnp.float32)
            for j in range(16):
                r = g * 16 + j
                acc = None
                for q in range(4):
                    sl_re = pl.ds(q * 16, 16)
                    sl_im = pl.ds(DIM + q * 16, 16)
                    s_r = sbuf[r, sl_re]
                    s_i = sbuf[r, sl_im]
                    o_r = obuf[r, sl_re]
                    o_i = obuf[r, sl_im]
                    a = o_r * w0[q] + o_i * w1[q]
                    b = o_i * w0[q] - o_r * w1[q]
                    t = s_r * a + s_i * b
                    acc = t if acc is None else acc + t
                for p in perms:
                    acc = acc + acc[p]
                out_acc = jnp.where(lane == j, acc, out_acc)
            out_v[pl.ds(co + g * 16, 16)] = out_acc
            return 0

        lax.fori_loop(0, _CHUNK // 16, group_body, 0)

    pltpu.sync_copy(out_v, out_hbm.at[pl.ds(base, _BPW)])


@jax.jit
def _run(s_idx, o_idx, fused, w_flat):
    mesh = plsc.VectorSubcoreMesh(core_axis_name="c", subcore_axis_name="s")
    f = pl.kernel(
        _sc_body,
        out_type=jax.ShapeDtypeStruct((BATCH,), jnp.float32),
        mesh=mesh,
        scratch_types=[
            pltpu.VMEM((_BPW,), jnp.int32),
            pltpu.VMEM((_BPW,), jnp.int32),
            pltpu.VMEM((2, _CHUNK, 2 * DIM), jnp.float32),
            pltpu.VMEM((2, _CHUNK, 2 * DIM), jnp.float32),
            pltpu.VMEM((2 * DIM,), jnp.float32),
            pltpu.VMEM((_BPW,), jnp.float32),
            pltpu.SemaphoreType.DMA,
        ],
        compiler_params=pltpu.CompilerParams(use_tc_tiling_on_sc=False),
    )
    return f(s_idx, o_idx, fused, w_flat)


_EB = 16384


def _fuse_body(re_ref, im_ref, out_ref):
    x = jnp.concatenate([re_ref[...], im_ref[...]], axis=0)
    ident = jnp.eye(2 * DIM, dtype=jnp.float32)
    out_ref[...] = jax.lax.dot_general(
        x, ident, (((0,), (0,)), ((), ())),
        preferred_element_type=jnp.float32)


@jax.jit
def _fuse(re_t, im_t):
    grid = (NUM_ENTITY + _EB - 1) // _EB
    return pl.pallas_call(
        _fuse_body,
        grid=(grid,),
        in_specs=[
            pl.BlockSpec((DIM, _EB), lambda j: (0, j)),
            pl.BlockSpec((DIM, _EB), lambda j: (0, j)),
        ],
        out_specs=pl.BlockSpec((_EB, 2 * DIM), lambda j: (j, 0)),
        out_shape=jax.ShapeDtypeStruct((NUM_ENTITY, 2 * DIM), jnp.float32),
    )(re_t, im_t)


def kernel(idxs, emb_re, emb_im, w):
    idxs = idxs.astype(jnp.int32)
    s_idx = idxs[:, 0]
    o_idx = idxs[:, 1]
    fused = _fuse(emb_re.T, emb_im.T)
    w_flat = w.reshape(-1)
    return _run(s_idx, o_idx, fused, w_flat)
